# baseline (speedup 1.0000x reference)
import functools

import jax
import jax.numpy as jnp
from jax import lax
from jax.experimental import pallas as pl
from jax.experimental.pallas import tpu as pltpu
from jax.experimental.pallas import tpu_sc as plsc

M = 16384
K = 8192
D = 256
TM = 256
TK = 1024
GM = M // TM
GK = K // TK

NC = 2
NS = 16
NW = NC * NS
ROWS_PER_W = M // NW
CHUNK = 128
NCHUNK = ROWS_PER_W // CHUNK


def _argmin_body(x_ref, e_ref, xn_ref, cn_ref, idx_ref, run_min, run_idx):
    k = pl.program_id(1)

    @pl.when(k == 0)
    def _init():
        run_min[...] = jnp.full((TM, 1), jnp.inf, jnp.float32)
        run_idx[...] = jnp.zeros((TM, 1), jnp.int32)

    x = x_ref[...]
    e = e_ref[...]
    mm = lax.dot_general(x, e, (((1,), (1,)), ((), ())),
                         preferred_element_type=jnp.float32)
    dist = (xn_ref[...] - 2.0 * mm) + cn_ref[...]
    min_d = jnp.min(dist, axis=1, keepdims=True)
    col = lax.broadcasted_iota(jnp.int32, (TM, TK), 1)
    local_arg = jnp.min(jnp.where(dist == min_d, col, jnp.int32(K)),
                        axis=1, keepdims=True)
    cand_idx = k * TK + local_arg
    better = min_d < run_min[...]
    run_idx[...] = jnp.where(better, cand_idx, run_idx[...])
    run_min[...] = jnp.where(better, min_d, run_min[...])

    @pl.when(k == GK - 1)
    def _out():
        idx_ref[...] = run_idx[...]


def _nearest_idx(flat, codebook, xnorm, cnorm):
    return pl.pallas_call(
        _argmin_body,
        grid=(GM, GK),
        in_specs=[
            pl.BlockSpec((TM, D), lambda m, k: (m, 0)),
            pl.BlockSpec((TK, D), lambda m, k: (k, 0)),
            pl.BlockSpec((TM, 1), lambda m, k: (m, 0)),
            pl.BlockSpec((1, TK), lambda m, k: (0, k)),
        ],
        out_specs=pl.BlockSpec((TM, 1), lambda m, k: (m, 0)),
        out_shape=jax.ShapeDtypeStruct((M, 1), jnp.int32),
        scratch_shapes=[
            pltpu.VMEM((TM, 1), jnp.float32),
            pltpu.VMEM((TM, 1), jnp.int32),
        ],
    )(flat, codebook, xnorm, cnorm)


@functools.cache
def _sc_gather_fn():
    @functools.partial(
        pl.kernel,
        mesh=plsc.VectorSubcoreMesh(core_axis_name="c", subcore_axis_name="s"),
        out_type=jax.ShapeDtypeStruct((M, D), jnp.float32),
        scratch_types=[
            pltpu.VMEM((NCHUNK, CHUNK), jnp.int32),
            pltpu.VMEM((CHUNK, D), jnp.float32),
            pltpu.SemaphoreType.DMA,
        ],
    )
    def _sc_gather(table_hbm, idx_hbm, out_hbm, idx_v, rows_v, sem):
        wid = lax.axis_index("s") * NC + lax.axis_index("c")
        pltpu.sync_copy(idx_hbm.at[pl.ds(wid * NCHUNK, NCHUNK)], idx_v)
        for j in range(NCHUNK):
            pltpu.async_copy(table_hbm.at[idx_v.at[j]], rows_v, sem).wait()
            pltpu.sync_copy(
                rows_v,
                out_hbm.at[pl.ds(wid * ROWS_PER_W + j * CHUNK, CHUNK)])

    return _sc_gather


def kernel(z, codebook):
    B, T, _ = z.shape
    flat = z.reshape(-1, D)
    xnorm = jnp.sum(flat * flat, axis=-1, keepdims=True)
    cnorm = jnp.sum(codebook * codebook, axis=-1)[None, :]
    idx = _nearest_idx(flat, codebook, xnorm, cnorm)
    q = _sc_gather_fn()(codebook, idx.reshape(M // CHUNK, CHUNK))
    q = q.reshape(B, T, D)
    return z + lax.stop_gradient(q - z)


# speedup vs baseline: 2.3021x; 2.3021x over previous
import functools

import jax
import jax.numpy as jnp
from jax import lax
from jax.experimental import pallas as pl
from jax.experimental.pallas import tpu as pltpu
from jax.experimental.pallas import tpu_sc as plsc

M = 16384
K = 8192
D = 256
TM = 256
TK = 8192
GM = M // TM
GK = K // TK

NC = 2
NS = 16
NW = NC * NS
ROWS_PER_W = M // NW
CHUNK = 128
NCHUNK = ROWS_PER_W // CHUNK


SUB = 1024
NSUB = TK // SUB


def _argmin_body(xs_ref, e_ref, xn_ref, cn_ref, col_ref, idx_ref,
                 run_min, run_idx):
    k = pl.program_id(1)

    @pl.when(k == 0)
    def _init():
        run_min[...] = jnp.full((TM, 1), jnp.inf, jnp.float32)
        run_idx[...] = jnp.zeros((TM, 1), jnp.float32)

    xs = xs_ref[...]
    xn = xn_ref[...]
    rm = run_min[...]
    ri = run_idx[...]
    for c in range(NSUB):
        e = e_ref[pl.ds(c * SUB, SUB), :]
        mm = lax.dot_general(xs, e, (((1,), (1,)), ((), ())),
                             preferred_element_type=jnp.float32)
        dist = (xn + mm) + cn_ref[:, pl.ds(c * SUB, SUB)]
        min_d = jnp.min(dist, axis=1, keepdims=True)
        cand_idx = jnp.min(
            jnp.where(dist == min_d, col_ref[:, pl.ds(c * SUB, SUB)],
                      jnp.float32(K)),
            axis=1, keepdims=True)
        better = min_d < rm
        ri = jnp.where(better, cand_idx, ri)
        rm = jnp.where(better, min_d, rm)
    run_min[...] = rm
    run_idx[...] = ri

    @pl.when(k == GK - 1)
    def _out():
        idx_ref[...] = ri.astype(jnp.int32)


def _nearest_idx(flat, codebook, xnorm, cnorm, cols):
    return pl.pallas_call(
        _argmin_body,
        grid=(GM, GK),
        in_specs=[
            pl.BlockSpec((TM, D), lambda m, k: (m, 0)),
            pl.BlockSpec((TK, D), lambda m, k: (k, 0)),
            pl.BlockSpec((TM, 1), lambda m, k: (m, 0)),
            pl.BlockSpec((1, TK), lambda m, k: (0, k)),
            pl.BlockSpec((1, TK), lambda m, k: (0, k)),
        ],
        out_specs=pl.BlockSpec((TM, 1), lambda m, k: (m, 0)),
        out_shape=jax.ShapeDtypeStruct((M, 1), jnp.int32),
        scratch_shapes=[
            pltpu.VMEM((TM, 1), jnp.float32),
            pltpu.VMEM((TM, 1), jnp.float32),
        ],
    )(flat, codebook, xnorm, cnorm, cols)


@functools.cache
def _sc_gather_fn():
    @functools.partial(
        pl.kernel,
        mesh=plsc.VectorSubcoreMesh(core_axis_name="c", subcore_axis_name="s"),
        out_type=jax.ShapeDtypeStruct((M, D), jnp.float32),
        scratch_types=[
            pltpu.VMEM((NCHUNK, CHUNK), jnp.int32),
            pltpu.VMEM((CHUNK, D), jnp.float32),
            pltpu.SemaphoreType.DMA,
        ],
    )
    def _sc_gather(table_hbm, idx_hbm, out_hbm, idx_v, rows_v, sem):
        wid = lax.axis_index("s") * NC + lax.axis_index("c")
        pltpu.sync_copy(idx_hbm.at[pl.ds(wid * NCHUNK, NCHUNK)], idx_v)
        for j in range(NCHUNK):
            pltpu.async_copy(table_hbm.at[idx_v.at[j]], rows_v, sem).wait()
            pltpu.sync_copy(
                rows_v,
                out_hbm.at[pl.ds(wid * ROWS_PER_W + j * CHUNK, CHUNK)])

    return _sc_gather


def kernel(z, codebook):
    B, T, _ = z.shape
    flat = z.reshape(-1, D)
    xnorm = jnp.sum(flat * flat, axis=-1, keepdims=True)
    cnorm = jnp.sum(codebook * codebook, axis=-1)[None, :]
    cols = jnp.arange(K, dtype=jnp.float32)[None, :]
    idx = _nearest_idx(-2.0 * flat, codebook, xnorm, cnorm, cols)
    q = _sc_gather_fn()(codebook, idx.reshape(M // CHUNK, CHUNK))
    q = q.reshape(B, T, D)
    return z + lax.stop_gradient(q - z)


# speedup vs baseline: 2.5300x; 1.0990x over previous
import functools

import jax
import jax.numpy as jnp
from jax import lax
from jax.experimental import pallas as pl
from jax.experimental.pallas import tpu as pltpu
from jax.experimental.pallas import tpu_sc as plsc

M = 16384
K = 8192
D = 256
TM = 1024
TK = 8192
GM = M // TM
GK = K // TK

NC = 2
NS = 16
NW = NC * NS
ROWS_PER_W = M // NW
CHUNK = 128
NCHUNK = ROWS_PER_W // CHUNK


SUB = 1024
NSUB = TK // SUB


def _argmin_body(xs_ref, e_ref, xn_ref, cn_ref, col_ref, idx_ref,
                 run_min, run_idx):
    k = pl.program_id(1)

    @pl.when(k == 0)
    def _init():
        run_min[...] = jnp.full((TM, 1), jnp.inf, jnp.float32)
        run_idx[...] = jnp.zeros((TM, 1), jnp.float32)

    xs = xs_ref[...] * jnp.float32(-2.0)
    xn = xn_ref[...]
    rm = run_min[...]
    ri = run_idx[...]
    for c in range(NSUB):
        e = e_ref[pl.ds(c * SUB, SUB), :]
        mm = lax.dot_general(xs, e, (((1,), (1,)), ((), ())),
                             preferred_element_type=jnp.float32)
        dist = (xn + mm) + cn_ref[:, pl.ds(c * SUB, SUB)]
        min_d = jnp.min(dist, axis=1, keepdims=True)
        cand_idx = jnp.min(
            jnp.where(dist == min_d, col_ref[:, pl.ds(c * SUB, SUB)],
                      jnp.float32(K)),
            axis=1, keepdims=True)
        better = min_d < rm
        ri = jnp.where(better, cand_idx, ri)
        rm = jnp.where(better, min_d, rm)
    run_min[...] = rm
    run_idx[...] = ri

    @pl.when(k == GK - 1)
    def _out():
        idx_ref[...] = ri.astype(jnp.int32)


def _nearest_idx(flat, codebook, xnorm, cnorm, cols):
    return pl.pallas_call(
        _argmin_body,
        grid=(GM, GK),
        in_specs=[
            pl.BlockSpec((TM, D), lambda m, k: (m, 0)),
            pl.BlockSpec((TK, D), lambda m, k: (k, 0)),
            pl.BlockSpec((TM, 1), lambda m, k: (m, 0)),
            pl.BlockSpec((1, TK), lambda m, k: (0, k)),
            pl.BlockSpec((1, TK), lambda m, k: (0, k)),
        ],
        out_specs=pl.BlockSpec((TM, 1), lambda m, k: (m, 0)),
        out_shape=jax.ShapeDtypeStruct((M, 1), jnp.int32),
        scratch_shapes=[
            pltpu.VMEM((TM, 1), jnp.float32),
            pltpu.VMEM((TM, 1), jnp.float32),
        ],
    )(flat, codebook, xnorm, cnorm, cols)


@functools.cache
def _sc_gather_fn():
    @functools.partial(
        pl.kernel,
        mesh=plsc.VectorSubcoreMesh(core_axis_name="c", subcore_axis_name="s"),
        out_type=jax.ShapeDtypeStruct((M, D), jnp.float32),
        scratch_types=[
            pltpu.VMEM((NCHUNK, CHUNK), jnp.int32),
            pltpu.VMEM((CHUNK, D), jnp.float32),
            pltpu.SemaphoreType.DMA,
        ],
    )
    def _sc_gather(table_hbm, idx_hbm, out_hbm, idx_v, rows_v, sem):
        wid = lax.axis_index("s") * NC + lax.axis_index("c")
        pltpu.sync_copy(idx_hbm.at[pl.ds(wid * NCHUNK, NCHUNK)], idx_v)
        for j in range(NCHUNK):
            pltpu.async_copy(table_hbm.at[idx_v.at[j]], rows_v, sem).wait()
            pltpu.sync_copy(
                rows_v,
                out_hbm.at[pl.ds(wid * ROWS_PER_W + j * CHUNK, CHUNK)])

    return _sc_gather


def kernel(z, codebook):
    B, T, _ = z.shape
    flat = z.reshape(-1, D)
    xnorm = jnp.sum(flat * flat, axis=-1, keepdims=True)
    cnorm = jnp.sum(codebook * codebook, axis=-1)[None, :]
    cols = jnp.arange(K, dtype=jnp.float32)[None, :]
    idx = _nearest_idx(flat, codebook, xnorm, cnorm, cols)
    q = _sc_gather_fn()(codebook, idx.reshape(M // CHUNK, CHUNK))
    q = q.reshape(B, T, D)
    return z + lax.stop_gradient(q - z)


# speedup vs baseline: 2.5752x; 1.0179x over previous
import functools

import jax
import jax.numpy as jnp
from jax import lax
from jax.experimental import pallas as pl
from jax.experimental.pallas import tpu as pltpu
from jax.experimental.pallas import tpu_sc as plsc

M = 16384
K = 8192
D = 256
TM = 2048
TK = 8192
GM = M // TM
GK = K // TK

NC = 2
NS = 16
NW = NC * NS
ROWS_PER_W = M // NW
CHUNK = 128
NCHUNK = ROWS_PER_W // CHUNK


SUB = 1024
NSUB = TK // SUB


def _argmin_body(xs_ref, e_ref, xn_ref, cn_ref, col_ref, idx_ref,
                 run_min, run_idx):
    k = pl.program_id(1)

    @pl.when(k == 0)
    def _init():
        run_min[...] = jnp.full((TM, 1), jnp.inf, jnp.float32)
        run_idx[...] = jnp.zeros((TM, 1), jnp.float32)

    xs = xs_ref[...] * jnp.float32(-2.0)
    xn = xn_ref[...]
    rm = run_min[...]
    ri = run_idx[...]
    for c in range(NSUB):
        e = e_ref[pl.ds(c * SUB, SUB), :]
        mm = lax.dot_general(xs, e, (((1,), (1,)), ((), ())),
                             preferred_element_type=jnp.float32)
        dist = (xn + mm) + cn_ref[:, pl.ds(c * SUB, SUB)]
        min_d = jnp.min(dist, axis=1, keepdims=True)
        cand_idx = jnp.min(
            jnp.where(dist == min_d, col_ref[:, pl.ds(c * SUB, SUB)],
                      jnp.float32(K)),
            axis=1, keepdims=True)
        better = min_d < rm
        ri = jnp.where(better, cand_idx, ri)
        rm = jnp.where(better, min_d, rm)
    run_min[...] = rm
    run_idx[...] = ri

    @pl.when(k == GK - 1)
    def _out():
        idx_ref[...] = ri.astype(jnp.int32)


def _nearest_idx(flat, codebook, xnorm, cnorm, cols):
    return pl.pallas_call(
        _argmin_body,
        grid=(GM, GK),
        in_specs=[
            pl.BlockSpec((TM, D), lambda m, k: (m, 0)),
            pl.BlockSpec((TK, D), lambda m, k: (k, 0)),
            pl.BlockSpec((TM, 1), lambda m, k: (m, 0)),
            pl.BlockSpec((1, TK), lambda m, k: (0, k)),
            pl.BlockSpec((1, TK), lambda m, k: (0, k)),
        ],
        out_specs=pl.BlockSpec((TM, 1), lambda m, k: (m, 0)),
        out_shape=jax.ShapeDtypeStruct((M, 1), jnp.int32),
        scratch_shapes=[
            pltpu.VMEM((TM, 1), jnp.float32),
            pltpu.VMEM((TM, 1), jnp.float32),
        ],
    )(flat, codebook, xnorm, cnorm, cols)


@functools.cache
def _sc_gather_fn():
    @functools.partial(
        pl.kernel,
        mesh=plsc.VectorSubcoreMesh(core_axis_name="c", subcore_axis_name="s"),
        out_type=jax.ShapeDtypeStruct((M, D), jnp.float32),
        scratch_types=[
            pltpu.VMEM((NCHUNK, CHUNK), jnp.int32),
            pltpu.VMEM((CHUNK, D), jnp.float32),
            pltpu.SemaphoreType.DMA,
        ],
    )
    def _sc_gather(table_hbm, idx_hbm, out_hbm, idx_v, rows_v, sem):
        wid = lax.axis_index("s") * NC + lax.axis_index("c")
        pltpu.sync_copy(idx_hbm.at[pl.ds(wid * NCHUNK, NCHUNK)], idx_v)
        for j in range(NCHUNK):
            pltpu.async_copy(table_hbm.at[idx_v.at[j]], rows_v, sem).wait()
            pltpu.sync_copy(
                rows_v,
                out_hbm.at[pl.ds(wid * ROWS_PER_W + j * CHUNK, CHUNK)])

    return _sc_gather


def kernel(z, codebook):
    B, T, _ = z.shape
    flat = z.reshape(-1, D)
    xnorm = jnp.sum(flat * flat, axis=-1, keepdims=True)
    cnorm = jnp.sum(codebook * codebook, axis=-1)[None, :]
    cols = jnp.arange(K, dtype=jnp.float32)[None, :]
    idx = _nearest_idx(flat, codebook, xnorm, cnorm, cols)
    q = _sc_gather_fn()(codebook, idx.reshape(M // CHUNK, CHUNK))
    q = q.reshape(B, T, D)
    return z + lax.stop_gradient(q - z)
